# baseline (speedup 1.0000x reference)
import jax
import jax.numpy as jnp
from jax.experimental import pallas as pl
from jax.experimental.pallas import tpu as pltpu


_TR = 2048


def _ffn_kernel(x_ref, w1_ref, b1_ref, w2_ref, b2_ref, o_ref):
    xb = x_ref[...].astype(jnp.bfloat16)
    h = jnp.dot(xb, w1_ref[...].astype(jnp.bfloat16),
                preferred_element_type=jnp.float32)
    h = jnp.maximum(h + b1_ref[...], 0.0)
    y = jnp.dot(h.astype(jnp.bfloat16), w2_ref[...].astype(jnp.bfloat16),
                preferred_element_type=jnp.float32)
    o_ref[...] = (y + b2_ref[...]).astype(o_ref.dtype)


def kernel(x, w1t, b1r, w2t, b2r):
    orig_shape = x.shape
    hidden_p = w1t.shape[0]
    ff_p = w1t.shape[1]
    rows = 1
    for d in orig_shape[:-1]:
        rows *= d
    x2 = x.reshape(rows, hidden_p)

    tr = _TR if rows % _TR == 0 else (256 if rows % 256 == 0 else 8)
    out = pl.pallas_call(
        _ffn_kernel,
        out_shape=jax.ShapeDtypeStruct((rows, hidden_p), x.dtype),
        grid=(rows // tr,),
        in_specs=[
            pl.BlockSpec((tr, hidden_p), lambda r: (r, 0)),
            pl.BlockSpec((hidden_p, ff_p), lambda r: (0, 0)),
            pl.BlockSpec((1, ff_p), lambda r: (0, 0)),
            pl.BlockSpec((ff_p, hidden_p), lambda r: (0, 0)),
            pl.BlockSpec((1, hidden_p), lambda r: (0, 0)),
        ],
        out_specs=pl.BlockSpec((tr, hidden_p), lambda r: (r, 0)),
        compiler_params=pltpu.CompilerParams(
            dimension_semantics=("parallel",),
        ),
    )(x2, w1t, b1r, w2t, b2r)
    return out.reshape(orig_shape)


# speedup vs baseline: 1.0238x; 1.0238x over previous
import jax
import jax.numpy as jnp
from jax.experimental import pallas as pl
from jax.experimental.pallas import tpu as pltpu


_TR = 1024


def _ffn_kernel(x_ref, w1_ref, b1_ref, w2_ref, b2_ref, o_ref):
    xb = x_ref[...].astype(jnp.bfloat16)
    h = jnp.dot(xb, w1_ref[...].astype(jnp.bfloat16),
                preferred_element_type=jnp.float32)
    h = jnp.maximum(h + b1_ref[...], 0.0)
    y = jnp.dot(h.astype(jnp.bfloat16), w2_ref[...].astype(jnp.bfloat16),
                preferred_element_type=jnp.float32)
    o_ref[...] = (y + b2_ref[...]).astype(o_ref.dtype)


def kernel(x, w1t, b1r, w2t, b2r):
    orig_shape = x.shape
    hidden_p = w1t.shape[0]
    ff_p = w1t.shape[1]
    rows = 1
    for d in orig_shape[:-1]:
        rows *= d
    x2 = x.reshape(rows, hidden_p)

    tr = _TR if rows % _TR == 0 else (256 if rows % 256 == 0 else 8)
    out = pl.pallas_call(
        _ffn_kernel,
        out_shape=jax.ShapeDtypeStruct((rows, hidden_p), x.dtype),
        grid=(rows // tr,),
        in_specs=[
            pl.BlockSpec((tr, hidden_p), lambda r: (r, 0)),
            pl.BlockSpec((hidden_p, ff_p), lambda r: (0, 0)),
            pl.BlockSpec((1, ff_p), lambda r: (0, 0)),
            pl.BlockSpec((ff_p, hidden_p), lambda r: (0, 0)),
            pl.BlockSpec((1, hidden_p), lambda r: (0, 0)),
        ],
        out_specs=pl.BlockSpec((tr, hidden_p), lambda r: (r, 0)),
        compiler_params=pltpu.CompilerParams(
            dimension_semantics=("parallel",),
        ),
    )(x2, w1t, b1r, w2t, b2r)
    return out.reshape(orig_shape)
